# baseline (speedup 1.0000x reference)
import functools

import jax
import jax.numpy as jnp
from jax.experimental import pallas as pl
from jax.experimental.pallas import tpu as pltpu
from jax.sharding import PartitionSpec as P

B, S, H = 1, 2048, 768
E, K, DFF = 8, 2, 1024
T = B * S
ROUTER_AUX_COEF = 0.001
ROUTER_Z_COEF = 0.001


def _router_body(x_ref, w_ref, b_ref, tp_ref, ti_ref, comb_ref, st_ref):
    tloc = x_ref.shape[0]
    xb = x_ref[...].astype(jnp.bfloat16)
    wb = w_ref[...].astype(jnp.bfloat16)
    logits = jax.lax.dot_general(
        xb, wb, (((1,), (0,)), ((), ())),
        preferred_element_type=jnp.float32) + b_ref[...][None, :]
    m = jnp.max(logits, axis=-1, keepdims=True)
    ex = jnp.exp(logits - m)
    z = jnp.sum(ex, axis=-1, keepdims=True)
    p = ex / z

    lane = jax.lax.broadcasted_iota(jnp.int32, (tloc, E), 1)
    v1 = jnp.max(p, axis=-1, keepdims=True)
    i1 = jnp.min(jnp.where(p == v1, lane, E), axis=-1, keepdims=True)
    p_m = jnp.where(lane == i1, -jnp.inf, p)
    v2 = jnp.max(p_m, axis=-1, keepdims=True)
    i2 = jnp.min(jnp.where(p_m == v2, lane, E), axis=-1, keepdims=True)

    tp_ref[...] = jnp.concatenate([v1, v2], axis=1)
    ti_ref[...] = jnp.concatenate([i1, i2], axis=1)
    onehot1 = (lane == i1).astype(jnp.float32)
    onehot2 = (lane == i2).astype(jnp.float32)
    comb_ref[...] = v1 * onehot1 + v2 * onehot2

    mask_sum = jnp.sum(onehot1 + onehot2, axis=0, keepdims=True)
    p_sum = jnp.sum(p, axis=0, keepdims=True)
    zm = jnp.maximum(v1, v2)
    lse = zm + jnp.log(jnp.exp(v1 - zm) + jnp.exp(v2 - zm))
    z_sum = jnp.sum(lse * lse, axis=0, keepdims=True)
    st_ref[...] = jnp.concatenate(
        [mask_sum, p_sum, z_sum, jnp.zeros((1, 7), jnp.float32)], axis=1)


def _ffn_body(x_ref, w1_ref, b1_ref, w2_ref, b2_ref, comb_ref, o_ref):
    tloc = x_ref.shape[0]
    e = pl.program_id(0)

    @pl.when(e == 0)
    def _():
        o_ref[...] = jnp.zeros_like(o_ref)

    lane = jax.lax.broadcasted_iota(jnp.int32, (tloc, E), 1)
    factor = jnp.sum(
        jnp.where(lane == e, comb_ref[...], 0.0), axis=1, keepdims=True)

    xb = x_ref[...].astype(jnp.bfloat16)
    w1b = w1_ref[0].astype(jnp.bfloat16)
    h = jax.lax.dot_general(
        xb, w1b, (((1,), (0,)), ((), ())),
        preferred_element_type=jnp.float32) + b1_ref[0]
    h = jax.nn.gelu(h)
    w2b = w2_ref[0].astype(jnp.bfloat16)
    y = jax.lax.dot_general(
        h.astype(jnp.bfloat16), w2b, (((1,), (0,)), ((), ())),
        preferred_element_type=jnp.float32) + b2_ref[0]
    o_ref[...] += factor * y


def _shard_fn(tokens, router_w, router_b, w1, b1, w2, b2):
    tloc = tokens.shape[0]
    top_probs, top_idx, combine, stats = pl.pallas_call(
        _router_body,
        out_shape=(
            jax.ShapeDtypeStruct((tloc, K), jnp.float32),
            jax.ShapeDtypeStruct((tloc, K), jnp.int32),
            jax.ShapeDtypeStruct((tloc, E), jnp.float32),
            jax.ShapeDtypeStruct((1, 24), jnp.float32),
        ),
    )(tokens, router_w, router_b)

    out = pl.pallas_call(
        _ffn_body,
        grid=(E,),
        in_specs=[
            pl.BlockSpec((tloc, H), lambda e: (0, 0)),
            pl.BlockSpec((1, H, DFF), lambda e: (e, 0, 0)),
            pl.BlockSpec((1, 1, DFF), lambda e: (e, 0, 0)),
            pl.BlockSpec((1, DFF, H), lambda e: (e, 0, 0)),
            pl.BlockSpec((1, 1, H), lambda e: (e, 0, 0)),
            pl.BlockSpec((tloc, E), lambda e: (0, 0)),
        ],
        out_specs=pl.BlockSpec((tloc, H), lambda e: (0, 0)),
        out_shape=jax.ShapeDtypeStruct((tloc, H), jnp.float32),
    )(tokens, w1, b1.reshape(E, 1, DFF), w2, b2.reshape(E, 1, H), combine)
    return out, top_probs, top_idx, stats


@jax.jit
def kernel(hidden_states, router_w, router_b, w1, b1, w2, b2):
    tokens = hidden_states.reshape(T, H)
    ndev = len(jax.devices())
    nsh = 2 if ndev >= 2 else 1
    mesh = jax.make_mesh((nsh,), ("x",))

    def _sh(a, spec):
        return jax.reshard(a, jax.sharding.NamedSharding(mesh, spec))

    f = jax.shard_map(
        _shard_fn,
        mesh=mesh,
        in_specs=(P("x"), P(), P(), P(), P(), P(), P()),
        out_specs=(P("x"), P("x"), P("x"), P("x")),
        check_vma=False,
    )
    out, top_probs, top_idx, stats = f(
        _sh(tokens, P("x")), _sh(router_w, P()), _sh(router_b, P()),
        _sh(w1, P()), _sh(b1.reshape(E, 1, DFF), P()),
        _sh(w2, P()), _sh(b2.reshape(E, 1, H), P()))

    s = jnp.sum(stats, axis=0)
    fraction = s[:E] / T
    mean_prob = s[E:2 * E] / T
    lbl = E * jnp.sum(fraction * mean_prob)
    zl = s[2 * E] / T
    aux_loss = lbl * ROUTER_AUX_COEF + zl * ROUTER_Z_COEF

    output = out.reshape(B, S, H)
    route_probs = top_probs.reshape(B, S, K)
    route_indices = top_idx.reshape(B, S, K)
    return (output, aux_loss, route_probs, route_indices)


# speedup vs baseline: 3.4437x; 3.4437x over previous
import functools

import jax
import jax.numpy as jnp
from jax.experimental import pallas as pl
from jax.experimental.pallas import tpu as pltpu
from jax.sharding import PartitionSpec as P

B, S, H = 1, 2048, 768
E, K, DFF = 8, 2, 1024
T = B * S
ROUTER_AUX_COEF = 0.001
ROUTER_Z_COEF = 0.001


def _router_body(x_ref, w_ref, b_ref, tp_ref, ti_ref, comb_ref, st_ref):
    tloc = x_ref.shape[0]
    xb = x_ref[...].astype(jnp.bfloat16)
    wb = w_ref[...].astype(jnp.bfloat16)
    logits = jax.lax.dot_general(
        xb, wb, (((1,), (0,)), ((), ())),
        preferred_element_type=jnp.float32) + b_ref[...][None, :]
    m = jnp.max(logits, axis=-1, keepdims=True)
    ex = jnp.exp(logits - m)
    z = jnp.sum(ex, axis=-1, keepdims=True)
    p = ex / z

    lane = jax.lax.broadcasted_iota(jnp.int32, (tloc, E), 1)
    v1 = jnp.max(p, axis=-1, keepdims=True)
    i1 = jnp.min(jnp.where(p == v1, lane, E), axis=-1, keepdims=True)
    p_m = jnp.where(lane == i1, -jnp.inf, p)
    v2 = jnp.max(p_m, axis=-1, keepdims=True)
    i2 = jnp.min(jnp.where(p_m == v2, lane, E), axis=-1, keepdims=True)

    tp_ref[...] = jnp.concatenate([v1, v2], axis=1)
    ti_ref[...] = jnp.concatenate([i1, i2], axis=1)
    onehot1 = (lane == i1).astype(jnp.float32)
    onehot2 = (lane == i2).astype(jnp.float32)
    comb_ref[...] = v1 * onehot1 + v2 * onehot2

    mask_sum = jnp.sum(onehot1 + onehot2, axis=0, keepdims=True)
    p_sum = jnp.sum(p, axis=0, keepdims=True)
    zm = jnp.maximum(v1, v2)
    lse = zm + jnp.log(jnp.exp(v1 - zm) + jnp.exp(v2 - zm))
    z_sum = jnp.sum(lse * lse, axis=0, keepdims=True)
    st_ref[...] = jnp.concatenate(
        [mask_sum, p_sum, z_sum, jnp.zeros((1, 7), jnp.float32)], axis=1)


def _ffn_body(x_ref, w1_ref, b1_ref, w2_ref, b2_ref, comb_ref, o_ref):
    tloc = x_ref.shape[0]
    e = pl.program_id(0)

    @pl.when(e == 0)
    def _():
        o_ref[...] = jnp.zeros_like(o_ref)

    lane = jax.lax.broadcasted_iota(jnp.int32, (tloc, E), 1)
    factor = jnp.sum(
        jnp.where(lane == e, comb_ref[...], 0.0), axis=1, keepdims=True)

    xb = x_ref[...].astype(jnp.bfloat16)
    w1b = w1_ref[0].astype(jnp.bfloat16)
    h = jax.lax.dot_general(
        xb, w1b, (((1,), (0,)), ((), ())),
        preferred_element_type=jnp.float32) + b1_ref[0]
    h = jax.nn.gelu(h)
    w2b = w2_ref[0].astype(jnp.bfloat16)
    y = jax.lax.dot_general(
        h.astype(jnp.bfloat16), w2b, (((1,), (0,)), ((), ())),
        preferred_element_type=jnp.float32) + b2_ref[0]
    o_ref[...] += factor * y


def _shard_fn(tokens_full, router_w, router_b, w1, b1, w2, b2):
    nsh = jax.lax.axis_size("x")
    tloc = tokens_full.shape[0] // nsh
    tokens = jax.lax.dynamic_slice_in_dim(
        tokens_full, jax.lax.axis_index("x") * tloc, tloc, 0)
    top_probs, top_idx, combine, stats = pl.pallas_call(
        _router_body,
        out_shape=(
            jax.ShapeDtypeStruct((tloc, K), jnp.float32),
            jax.ShapeDtypeStruct((tloc, K), jnp.int32),
            jax.ShapeDtypeStruct((tloc, E), jnp.float32),
            jax.ShapeDtypeStruct((1, 24), jnp.float32),
        ),
    )(tokens, router_w, router_b)

    out = pl.pallas_call(
        _ffn_body,
        grid=(E,),
        in_specs=[
            pl.BlockSpec((tloc, H), lambda e: (0, 0)),
            pl.BlockSpec((1, H, DFF), lambda e: (e, 0, 0)),
            pl.BlockSpec((1, 1, DFF), lambda e: (e, 0, 0)),
            pl.BlockSpec((1, DFF, H), lambda e: (e, 0, 0)),
            pl.BlockSpec((1, 1, H), lambda e: (e, 0, 0)),
            pl.BlockSpec((tloc, E), lambda e: (0, 0)),
        ],
        out_specs=pl.BlockSpec((tloc, H), lambda e: (0, 0)),
        out_shape=jax.ShapeDtypeStruct((tloc, H), jnp.float32),
    )(tokens, w1, b1.reshape(E, 1, DFF), w2, b2.reshape(E, 1, H), combine)
    return out, top_probs, top_idx, stats


@jax.jit
def kernel(hidden_states, router_w, router_b, w1, b1, w2, b2):
    tokens = hidden_states.reshape(T, H)
    ndev = len(jax.devices())
    nsh = 2 if ndev >= 2 else 1
    mesh = jax.make_mesh((nsh,), ("x",))
    f = jax.shard_map(
        _shard_fn,
        mesh=mesh,
        in_specs=(P(), P(), P(), P(), P(), P(), P()),
        out_specs=(P("x"), P("x"), P("x"), P("x")),
        check_vma=False,
    )
    out, top_probs, top_idx, stats = f(
        tokens, router_w, router_b, w1, b1.reshape(E, 1, DFF),
        w2, b2.reshape(E, 1, H))

    s = jnp.sum(stats, axis=0)
    fraction = s[:E] / T
    mean_prob = s[E:2 * E] / T
    lbl = E * jnp.sum(fraction * mean_prob)
    zl = s[2 * E] / T
    aux_loss = lbl * ROUTER_AUX_COEF + zl * ROUTER_Z_COEF

    output = out.reshape(B, S, H)
    route_probs = top_probs.reshape(B, S, K)
    route_indices = top_idx.reshape(B, S, K)
    return (output, aux_loss, route_probs, route_indices)


# speedup vs baseline: 4.5105x; 1.3098x over previous
import functools

import jax
import jax.numpy as jnp
from jax import lax
from jax.experimental import pallas as pl
from jax.experimental.pallas import tpu as pltpu
from jax.experimental.pallas import tpu_sc as plsc

B, S, H = 1, 2048, 768
E, K, DFF = 8, 2, 1024
T = B * S
NP = T * K
BT = 128
NB = NP // BT + E
P = NB * BT
ROUTER_AUX_COEF = 0.001
ROUTER_Z_COEF = 0.001


def _excl_cumsum0(x):
    c = x
    s = 1
    n = x.shape[0]
    while s < n:
        c = c + jnp.concatenate(
            [jnp.zeros((s, x.shape[1]), x.dtype), c[:-s, :]], axis=0)
        s *= 2
    return c - x


def _incl_cumsum1(x):
    c = x
    s = 1
    n = x.shape[1]
    while s < n:
        c = c + jnp.concatenate(
            [jnp.zeros((x.shape[0], s), x.dtype), c[:, :-s]], axis=1)
        s *= 2
    return c


def _router_body(x_ref, w_ref, b_ref,
                 tp_ref, ti_ref, aux_ref, p0_ref, p1_ref, bm_ref):
    xb = x_ref[...].astype(jnp.bfloat16)
    wb = w_ref[...].astype(jnp.bfloat16)
    logits = lax.dot_general(
        xb, wb, (((1,), (0,)), ((), ())),
        preferred_element_type=jnp.float32) + b_ref[...][None, :]
    m = jnp.max(logits, axis=-1, keepdims=True)
    ex = jnp.exp(logits - m)
    z = jnp.sum(ex, axis=-1, keepdims=True)
    p = ex / z

    lane = lax.broadcasted_iota(jnp.int32, (T, E), 1)
    v1 = jnp.max(p, axis=-1, keepdims=True)
    i1 = jnp.min(jnp.where(p == v1, lane, E), axis=-1, keepdims=True)
    p_m = jnp.where(lane == i1, -jnp.inf, p)
    v2 = jnp.max(p_m, axis=-1, keepdims=True)
    i2 = jnp.min(jnp.where(p_m == v2, lane, E), axis=-1, keepdims=True)

    tp_ref[...] = jnp.concatenate([v1, v2], axis=1)
    ti_ref[...] = jnp.concatenate([i1, i2], axis=1)
    oh1 = (lane == i1).astype(jnp.int32)
    oh2 = (lane == i2).astype(jnp.int32)

    mask = (oh1 + oh2).astype(jnp.float32)
    fraction = jnp.mean(mask, axis=0, keepdims=True)
    mean_prob = jnp.mean(p, axis=0, keepdims=True)
    lbl = E * jnp.sum(fraction * mean_prob, axis=1, keepdims=True)
    zm = jnp.maximum(v1, v2)
    lse = zm + jnp.log(jnp.exp(v1 - zm) + jnp.exp(v2 - zm))
    zl = jnp.mean(lse * lse, axis=0, keepdims=True)
    aux_ref[...] = lbl * ROUTER_AUX_COEF + zl * ROUTER_Z_COEF

    c0 = _excl_cumsum0(oh1)
    c1 = _excl_cumsum0(oh2)
    cnt0 = jnp.sum(oh1, axis=0, keepdims=True)
    cnt_t = cnt0 + jnp.sum(oh2, axis=0, keepdims=True)
    nb = (cnt_t + (BT - 1)) // BT
    end_blk = _incl_cumsum1(nb)
    start_blk = end_blk - nb
    group_start = start_blk * BT
    p0_ref[...] = jnp.sum(
        jnp.where(lane == i1, group_start + c0, 0), axis=1, keepdims=True)
    p1_ref[...] = jnp.sum(
        jnp.where(lane == i2, group_start + cnt0 + c1, 0),
        axis=1, keepdims=True)

    b_iota = lax.broadcasted_iota(jnp.int32, (NB, E), 0)
    bm = jnp.sum((end_blk <= b_iota).astype(jnp.int32), axis=1, keepdims=True)
    bm_ref[...] = jnp.minimum(bm, E - 1)


def _grouped_ffn_body(bm_ref, x_ref, w1_ref, b1_ref, w2_ref, b2_ref, o_ref):
    xb = x_ref[...].astype(jnp.bfloat16)
    w1b = w1_ref[0].astype(jnp.bfloat16)
    h = lax.dot_general(
        xb, w1b, (((1,), (0,)), ((), ())),
        preferred_element_type=jnp.float32) + b1_ref[0]
    h = jax.nn.gelu(h)
    w2b = w2_ref[0].astype(jnp.bfloat16)
    y = lax.dot_general(
        h.astype(jnp.bfloat16), w2b, (((1,), (0,)), ((), ())),
        preferred_element_type=jnp.float32) + b2_ref[0]
    o_ref[...] = y


def _make_dispatch():
    info = plsc.get_sparse_core_info()
    nw = info.num_cores * info.num_subcores
    cpw = NP // nw
    mesh = plsc.VectorSubcoreMesh(core_axis_name="c", subcore_axis_name="s")

    @functools.partial(
        pl.kernel, mesh=mesh,
        out_type=jax.ShapeDtypeStruct((P, H), jnp.float32),
        scratch_types=[
            pltpu.VMEM((cpw,), jnp.int32),
            pltpu.VMEM((cpw,), jnp.int32),
            pltpu.VMEM((cpw, H), jnp.float32),
            pltpu.SemaphoreType.DMA,
        ],
    )
    def dispatch(tok_hbm, pos_hbm, xs_hbm, pos_v, tix_v, rows_v, sem0):
        wid = lax.axis_index("s") * info.num_cores + lax.axis_index("c")
        base = wid * cpw
        tbase = lax.rem(base, T)
        pltpu.sync_copy(pos_hbm.at[pl.ds(base, cpw)], pos_v)
        for j in range(cpw // 16):
            tix_v[pl.ds(j * 16, 16)] = lax.iota(jnp.int32, 16) + (
                tbase + j * 16)
        pltpu.async_copy(tok_hbm.at[tix_v], rows_v, sem0).wait()
        pltpu.async_copy(rows_v, xs_hbm.at[pos_v], sem0).wait()

    return dispatch


def _wadd_body(y0_ref, y1_ref, tp_ref, o_ref):
    tp = tp_ref[...]
    o_ref[...] = y0_ref[...] * tp[:, :1] + y1_ref[...] * tp[:, 1:2]


def _make_combine():
    info = plsc.get_sparse_core_info()
    nw = info.num_cores * info.num_subcores
    tpw = T // nw
    mesh = plsc.VectorSubcoreMesh(core_axis_name="c", subcore_axis_name="s")

    @functools.partial(
        pl.kernel, mesh=mesh,
        out_type=[
            jax.ShapeDtypeStruct((T, H), jnp.float32),
            jax.ShapeDtypeStruct((T, H), jnp.float32),
        ],
        scratch_types=[
            pltpu.VMEM((tpw,), jnp.int32),
            pltpu.VMEM((tpw,), jnp.int32),
            pltpu.VMEM((tpw, H), jnp.float32),
            pltpu.VMEM((tpw, H), jnp.float32),
            pltpu.SemaphoreType.DMA,
            pltpu.SemaphoreType.DMA,
        ],
    )
    def combine(y_hbm, pos_hbm, y0g_hbm, y1g_hbm,
                p0_v, p1_v, y0_v, y1_v, sem0, sem1):
        wid = lax.axis_index("s") * info.num_cores + lax.axis_index("c")
        tb = wid * tpw
        pltpu.sync_copy(pos_hbm.at[pl.ds(tb, tpw)], p0_v)
        pltpu.sync_copy(pos_hbm.at[pl.ds(T + tb, tpw)], p1_v)
        g0 = pltpu.async_copy(y_hbm.at[p0_v], y0_v, sem0)
        g1 = pltpu.async_copy(y_hbm.at[p1_v], y1_v, sem1)
        g0.wait()
        g1.wait()
        pltpu.sync_copy(y0_v, y0g_hbm.at[pl.ds(tb, tpw)])
        pltpu.sync_copy(y1_v, y1g_hbm.at[pl.ds(tb, tpw)])

    return combine


@jax.jit
def kernel(hidden_states, router_w, router_b, w1, b1, w2, b2):
    tokens = hidden_states.reshape(T, H)

    top_probs, top_idx, aux, pos0, pos1, blkmap = pl.pallas_call(
        _router_body,
        out_shape=(
            jax.ShapeDtypeStruct((T, K), jnp.float32),
            jax.ShapeDtypeStruct((T, K), jnp.int32),
            jax.ShapeDtypeStruct((1, 1), jnp.float32),
            jax.ShapeDtypeStruct((T, 1), jnp.int32),
            jax.ShapeDtypeStruct((T, 1), jnp.int32),
            jax.ShapeDtypeStruct((NB, 1), jnp.int32),
        ),
    )(tokens, router_w, router_b)

    posflat = jnp.concatenate([pos0.reshape(T), pos1.reshape(T)])

    x_sorted = _make_dispatch()(tokens, posflat)

    grid_spec = pltpu.PrefetchScalarGridSpec(
        num_scalar_prefetch=1,
        grid=(NB,),
        in_specs=[
            pl.BlockSpec((BT, H), lambda b, m: (b, 0)),
            pl.BlockSpec((1, H, DFF), lambda b, m: (m[b], 0, 0)),
            pl.BlockSpec((1, 1, DFF), lambda b, m: (m[b], 0, 0)),
            pl.BlockSpec((1, DFF, H), lambda b, m: (m[b], 0, 0)),
            pl.BlockSpec((1, 1, H), lambda b, m: (m[b], 0, 0)),
        ],
        out_specs=pl.BlockSpec((BT, H), lambda b, m: (b, 0)),
    )
    y_sorted = pl.pallas_call(
        _grouped_ffn_body,
        grid_spec=grid_spec,
        out_shape=jax.ShapeDtypeStruct((P, H), jnp.float32),
    )(blkmap.reshape(NB), x_sorted, w1, b1.reshape(E, 1, DFF),
      w2, b2.reshape(E, 1, H))

    y0g, y1g = _make_combine()(y_sorted, posflat)

    out = pl.pallas_call(
        _wadd_body,
        out_shape=jax.ShapeDtypeStruct((T, H), jnp.float32),
    )(y0g, y1g, top_probs)

    output = out.reshape(B, S, H)
    aux_loss = aux[0, 0]
    route_probs = top_probs.reshape(B, S, K)
    route_indices = top_idx.reshape(B, S, K)
    return (output, aux_loss, route_probs, route_indices)
